# baseline (speedup 1.0000x reference)
import functools

import jax
import jax.numpy as jnp
import numpy as np
from jax import lax
from jax.experimental import pallas as pl
from jax.experimental.pallas import tpu as pltpu
from jax.experimental.pallas import tpu_sc as plsc

N_NODES = 10000
NUMS = [5000, 3000, 2000]
NUM_HIDDEN = 64
NUM_CLASSES = 16
NUM_LAYERS = 2
HEADS = [2, 2, 1]
N_GRAPHS = 2
N_TYPES = 3
E_EDGES = 160000
NEG_SLOPE = 0.2

NC, NS, LANES = 2, 16, 16
NW = NC * NS

_TYPE_OF = np.concatenate([np.full(NUMS[k], k, np.int32) for k in range(N_TYPES)])


def _mm_bias(a, b, bias):
    def body(a_ref, b_ref, c_ref, o_ref):
        o_ref[...] = jnp.dot(a_ref[...], b_ref[...],
                             preferred_element_type=jnp.float32) + c_ref[...]
    return pl.pallas_call(
        body,
        out_shape=jax.ShapeDtypeStruct((a.shape[0], b.shape[1]), jnp.float32),
    )(a, b, bias)


def _pack_bf16(e, o):
    def body(er, orr, outr):
        ue = lax.bitcast_convert_type(er[...], jnp.uint32)
        uo = lax.bitcast_convert_type(orr[...], jnp.uint32)
        pe = (ue + jnp.uint32(0x8000)) >> 16
        po = (uo + jnp.uint32(0x8000)) & jnp.uint32(0xffff0000)
        outr[...] = lax.bitcast_convert_type(pe | po, jnp.int32)
    return pl.pallas_call(
        body, out_shape=jax.ShapeDtypeStruct(e.shape, jnp.int32))(e, o)


def _combine_hidden(acc0, acc1, dp0, dp1, b0, b1, wv, hres, H, D, use_res):
    P = H * D

    def body(*refs):
        if use_res:
            a0, a1, d0, d1, b0r, b1r, wvr, hr, o = refs
        else:
            a0, a1, d0, d1, b0r, b1r, wvr, o = refs
            hr = None
        outs = []
        for g, (a, d, br) in enumerate(((a0, d0, b0r), (a1, d1, b1r))):
            invs = []
            for h in range(H):
                den = jnp.sum(d[h * NW:(h + 1) * NW, :], axis=0, keepdims=True)
                invs.append(jnp.broadcast_to(1.0 / (den + 1e-16), (D, den.shape[1])))
            inv = jnp.concatenate(invs, axis=0)
            x = a[...] * inv + br[...]
            if hr is not None:
                x = x + hr[...]
            x = jnp.where(x > 0, x, jnp.exp(jnp.minimum(x, 0.0)) - 1.0)
            outs.append(x * jnp.broadcast_to(wvr[g:g + 1, :], x.shape))
        o[...] = outs[0] + outs[1]

    args = [acc0, acc1, dp0, dp1, b0, b1, wv] + ([hres] if use_res else [])
    return pl.pallas_call(
        body,
        out_shape=jax.ShapeDtypeStruct((P, N_NODES), jnp.float32),
    )(*args)


def _combine_final(acc0, acc1, dp0, dp1, b0, b1, res0, res1, wv):
    def body(a0, a1, d0, d1, b0r, b1r, r0, r1, wvr, o):
        outs = []
        for g, (a, d, br, rr) in enumerate(((a0, d0, b0r, r0), (a1, d1, b1r, r1))):
            den = jnp.sum(d[...], axis=0, keepdims=True)
            inv = jnp.broadcast_to(1.0 / (den + 1e-16), (NUM_CLASSES, den.shape[1]))
            acc = a[0:NUM_CLASSES, :] + a[NUM_CLASSES:2 * NUM_CLASSES, :]
            x = acc * inv + br[...] + rr[...]
            outs.append(x * jnp.broadcast_to(wvr[g:g + 1, :], x.shape))
        o[...] = outs[0] + outs[1]

    return pl.pallas_call(
        body,
        out_shape=jax.ShapeDtypeStruct((NUM_CLASSES, N_NODES), jnp.float32),
    )(acc0, acc1, dp0, dp1, b0, b1, res0, res1, wv)


def _sc_mesh():
    return plsc.VectorSubcoreMesh(core_axis_name="c", subcore_axis_name="s",
                                  num_cores=NC, num_subcores=NS)


_SC_PARAMS = pltpu.CompilerParams(needs_layout_passes=False)


@functools.partial(jax.jit, static_argnames=("H",))
def _sc_alpha(elr, src, dst, H):
    EPT = E_EDGES // NW
    EPTP = ((EPT + LANES - 1) // LANES) * LANES
    NIT = EPTP // LANES

    scratch = ([pltpu.VMEM((N_NODES,), jnp.float32)] * (3 * H)
               + [pltpu.VMEM((EPTP,), jnp.int32)] * 2
               + [pltpu.VMEM((EPTP,), jnp.float32)] * H)

    @functools.partial(
        pl.kernel,
        out_type=(jax.ShapeDtypeStruct((H * E_EDGES,), jnp.float32),
                  jax.ShapeDtypeStruct((H * NW * N_NODES,), jnp.float32)),
        mesh=_sc_mesh(),
        scratch_types=scratch,
        compiler_params=_SC_PARAMS,
    )
    def k(elr_h, src_h, dst_h, au_h, dp_h, *scr):
        el_loc = scr[0:H]
        er_loc = scr[H:2 * H]
        den = scr[2 * H:3 * H]
        srcb, dstb = scr[3 * H], scr[3 * H + 1]
        aub = scr[3 * H + 2:3 * H + 2 + H]

        wid = lax.axis_index("c") * NS + lax.axis_index("s")
        base = wid * EPT
        for h in range(H):
            pltpu.sync_copy(elr_h.at[pl.ds(h * N_NODES, N_NODES)], el_loc[h])
            pltpu.sync_copy(elr_h.at[pl.ds((H + h) * N_NODES, N_NODES)], er_loc[h])
        pltpu.sync_copy(src_h.at[pl.ds(base, EPT)], srcb.at[pl.ds(0, EPT)])
        pltpu.sync_copy(dst_h.at[pl.ds(base, EPT)], dstb.at[pl.ds(0, EPT)])

        def zero_body(i, _):
            for h in range(H):
                den[h][pl.ds(i * LANES, LANES)] = jnp.zeros((LANES,), jnp.float32)
            return 0
        lax.fori_loop(0, N_NODES // LANES, zero_body, 0)

        if EPTP != EPT:
            lane = lax.iota(jnp.int32, LANES)
            keep = lane < (LANES - (EPTP - EPT))
            tail = pl.ds(EPTP - LANES, LANES)
            srcb[tail] = jnp.where(keep, srcb[tail], 0)
            dstb[tail] = jnp.where(keep, dstb[tail], 0)

        lane = lax.iota(jnp.int32, LANES)

        def body(i, _):
            kofs = i * LANES
            s16 = srcb[pl.ds(kofs, LANES)]
            d16 = dstb[pl.ds(kofs, LANES)]
            valid = (kofs + lane) < EPT
            for h in range(H):
                ev = plsc.load_gather(el_loc[h], [s16])
                rv = plsc.load_gather(er_loc[h], [d16])
                e = ev + rv
                e = jnp.where(e > 0, e, NEG_SLOPE * e)
                a = jnp.exp(e)
                aub[h][pl.ds(kofs, LANES)] = a
                plsc.addupdate_scatter(den[h], [d16], a, mask=valid)
            return 0
        lax.fori_loop(0, NIT, body, 0)

        for h in range(H):
            pltpu.sync_copy(aub[h].at[pl.ds(0, EPT)],
                            au_h.at[pl.ds(h * E_EDGES + base, EPT)])
            pltpu.sync_copy(den[h],
                            dp_h.at[pl.ds((h * NW + wid) * N_NODES, N_NODES)])

    return k(elr, src, dst)


@functools.partial(jax.jit, static_argnames=("H", "D", "esplit", "packed"))
def _sc_aggregate(featT, au, src, dst, H, D, esplit, packed):
    P = H * D
    PPT = P * esplit // NW
    NG = P // PPT
    EP = E_EDGES // esplit
    CH = 4000
    NCH = EP // CH
    UNROLL = 10
    NPHYS = PPT // 2 if packed else PPT
    fdt = jnp.int32 if packed else jnp.float32

    NACC = 2 * PPT if packed else PPT
    scratch = (
        [pltpu.VMEM((N_NODES,), fdt) for _ in range(NPHYS)]
        + [pltpu.VMEM((N_NODES,), jnp.float32) for _ in range(NACC)]
        + [pltpu.VMEM((CH,), jnp.int32) for _ in range(2)]
        + [pltpu.VMEM((CH,), jnp.int32) for _ in range(2)]
        + [pltpu.VMEM((CH,), jnp.float32) for _ in range(2)]
        + [pltpu.SemaphoreType.DMA for _ in range(2)]
    )

    @functools.partial(
        pl.kernel,
        out_type=jax.ShapeDtypeStruct((esplit * P * N_NODES,), jnp.float32),
        mesh=_sc_mesh(),
        scratch_types=scratch,
        compiler_params=_SC_PARAMS,
    )
    def k(featT_h, au_h, src_h, dst_h, out_h, *scr):
        fps = scr[0:NPHYS]
        aps = scr[NPHYS:NPHYS + PPT]
        apsB = scr[NPHYS + PPT:NPHYS + NACC]
        o = NPHYS + NACC
        srccs = scr[o:o + 2]
        dstcs = scr[o + 2:o + 4]
        aucs = scr[o + 4:o + 6]
        isems = scr[o + 6:o + 8]

        wid = lax.axis_index("c") * NS + lax.axis_index("s")
        g = wid % NG
        s = wid // NG
        pbase = g * PPT
        h = pbase // D
        fbase = g * NPHYS
        for p in range(NPHYS):
            pltpu.sync_copy(
                featT_h.at[pl.ds((fbase + p) * N_NODES, N_NODES)], fps[p])

        def zero_body(i, _):
            for r in (list(aps) + list(apsB)):
                r[pl.ds(i * LANES, LANES)] = jnp.zeros((LANES,), jnp.float32)
            return 0
        lax.fori_loop(0, N_NODES // LANES, zero_body, 0)

        ebase = s * EP

        def issue_idx(c, b):
            off = ebase + c * CH
            pltpu.async_copy(src_h.at[pl.ds(off, CH)], srccs[b], isems[b])
            pltpu.async_copy(dst_h.at[pl.ds(off, CH)], dstcs[b], isems[b])
            pltpu.async_copy(au_h.at[pl.ds(h * E_EDGES + off, CH)],
                             aucs[b], isems[b])

        def drain_idx(c, b):
            off = ebase + c * CH
            pltpu.make_async_copy(src_h.at[pl.ds(off, CH)], srccs[b],
                                  isems[b]).wait()
            pltpu.make_async_copy(dst_h.at[pl.ds(off, CH)], dstcs[b],
                                  isems[b]).wait()
            pltpu.make_async_copy(au_h.at[pl.ds(h * E_EDGES + off, CH)],
                                  aucs[b], isems[b]).wait()

        issue_idx(0, 0)
        issue_idx(1, 1)

        def chunk_pair(j, _):
            for b in range(2):
                c = 2 * j + b
                drain_idx(c, b)

                def body(i, _):
                    for u in range(UNROLL):
                        kofs = i * (LANES * UNROLL) + u * LANES
                        s16 = srccs[b][pl.ds(kofs, LANES)]
                        d16 = dstcs[b][pl.ds(kofs, LANES)]
                        a16 = aucs[b][pl.ds(kofs, LANES)]
                        if packed:
                            tgt = aps if u % 2 == 0 else apsB
                            for p in range(NPHYS):
                                w = plsc.load_gather(fps[p], [s16])
                                ve = plsc.bitcast(w << 16, jnp.float32)
                                vo = plsc.bitcast(w & jnp.int32(-65536),
                                                  jnp.float32)
                                plsc.addupdate_scatter(tgt[2 * p], [d16],
                                                       ve * a16)
                                plsc.addupdate_scatter(tgt[2 * p + 1], [d16],
                                                       vo * a16)
                        else:
                            for p in range(PPT):
                                v = plsc.load_gather(fps[p], [s16])
                                plsc.addupdate_scatter(aps[p], [d16], v * a16)
                    return 0
                lax.fori_loop(0, CH // (LANES * UNROLL), body, 0)

                @pl.when(c + 2 < NCH)
                def _():
                    issue_idx(c + 2, b)
            return 0
        lax.fori_loop(0, NCH // 2, chunk_pair, 0)

        if packed:
            def merge_body(i, _):
                sl = pl.ds(i * LANES, LANES)
                for p in range(PPT):
                    aps[p][sl] = aps[p][sl] + apsB[p][sl]
                return 0
            lax.fori_loop(0, N_NODES // LANES, merge_body, 0)

        for p in range(PPT):
            pltpu.sync_copy(aps[p],
                            out_h.at[pl.ds((s * P + pbase + p) * N_NODES,
                                           N_NODES)])

    return k(featT, au, src, dst)


def kernel(feat0, feat1, feat2, edge_index_g0, edge_index_g1, params):
    graphs = [(edge_index_g0[0].astype(jnp.int32), edge_index_g0[1].astype(jnp.int32)),
              (edge_index_g1[0].astype(jnp.int32), edge_index_g1[1].astype(jnp.int32))]
    w = jax.nn.softmax(params['rel_weights'], axis=2)
    type_of = jnp.asarray(_TYPE_OF)

    hs = []
    for i, f in enumerate((feat0, feat1, feat2)):
        p = params['fc'][i]
        hs.append(_mm_bias(p['W'].T, f.T, p['b'][:, None]))
    hT = jnp.concatenate(hs, axis=1)

    D = NUM_HIDDEN
    for l in range(NUM_LAYERS):
        H = HEADS[l]
        P = H * D
        perm = np.concatenate([np.arange(0, P, 2), np.arange(1, P, 2)])
        cols = []
        for i in range(N_GRAPHS):
            cols.append(params['gat'][i][l]['W'][:, perm])
        for i in range(N_GRAPHS):
            pg = params['gat'][i][l]
            al = [pg['W'][:, h * D:(h + 1) * D] @ pg['attn_l'][h] for h in range(H)]
            ar = [pg['W'][:, h * D:(h + 1) * D] @ pg['attn_r'][h] for h in range(H)]
            cols.append(jnp.stack(al + ar, axis=1))
        A = jnp.concatenate(cols, axis=1).T
        big = _mm_bias(A, hT, jnp.zeros((A.shape[0], 1), jnp.float32))
        accs, dps = [], []
        for i in range(N_GRAPHS):
            src, dst = graphs[i]
            featE = big[i * P:i * P + P // 2]
            featO = big[i * P + P // 2:(i + 1) * P]
            elr = big[2 * P + 2 * H * i:2 * P + 2 * H * (i + 1)]
            au, dp = _sc_alpha(elr.reshape(-1), src, dst, H)
            packedT = _pack_bf16(featE, featO)
            acc = _sc_aggregate(packedT.reshape(-1), au, src, dst, H, D, 1,
                                packed=True)
            accs.append(acc.reshape(P, N_NODES))
            dps.append(dp.reshape(H * NW, N_NODES))
        wv = jnp.stack([w[type_of, l, i] for i in range(N_GRAPHS)])
        b0 = params['gat'][0][l]['bias'][:, None]
        b1 = params['gat'][1][l]['bias'][:, None]
        hT = _combine_hidden(accs[0], accs[1], dps[0], dps[1], b0, b1, wv,
                             hT, H, D, use_res=(l > 0))

    H, Df = HEADS[-1], NUM_CLASSES
    cols = []
    for i in range(N_GRAPHS):
        cols.append(params['gat'][i][NUM_LAYERS]['W'])
    for i in range(N_GRAPHS):
        cols.append(params['gat'][i][NUM_LAYERS]['res_W'])
    for i in range(N_GRAPHS):
        pg = params['gat'][i][NUM_LAYERS]
        cols.append(jnp.stack([pg['W'] @ pg['attn_l'][0],
                               pg['W'] @ pg['attn_r'][0]], axis=1))
    A = jnp.concatenate(cols, axis=1).T
    big = _mm_bias(A, hT, jnp.zeros((A.shape[0], 1), jnp.float32))
    accs, dps = [], []
    for i in range(N_GRAPHS):
        src, dst = graphs[i]
        featT = big[i * Df:(i + 1) * Df]
        elr = big[4 * Df + 2 * i:4 * Df + 2 * (i + 1)]
        au, dp = _sc_alpha(elr.reshape(-1), src, dst, H)
        acc = _sc_aggregate(featT.reshape(-1), au, src, dst, H, Df, 2,
                            packed=False)
        accs.append(acc.reshape(2 * Df, N_NODES))
        dps.append(dp.reshape(H * NW, N_NODES))
    wv = jnp.stack([w[type_of, NUM_LAYERS, i] for i in range(N_GRAPHS)])
    b0 = params['gat'][0][NUM_LAYERS]['bias'][:, None]
    b1 = params['gat'][1][NUM_LAYERS]['bias'][:, None]
    res0 = big[2 * Df:3 * Df]
    res1 = big[3 * Df:4 * Df]
    logitsT = _combine_final(accs[0], accs[1], dps[0], dps[1], b0, b1,
                             res0, res1, wv)
    return logitsT.T


# speedup vs baseline: 1.0477x; 1.0477x over previous
import functools

import jax
import jax.numpy as jnp
import numpy as np
from jax import lax
from jax.experimental import pallas as pl
from jax.experimental.pallas import tpu as pltpu
from jax.experimental.pallas import tpu_sc as plsc

N_NODES = 10000
NUMS = [5000, 3000, 2000]
NUM_HIDDEN = 64
NUM_CLASSES = 16
NUM_LAYERS = 2
HEADS = [2, 2, 1]
N_GRAPHS = 2
N_TYPES = 3
E_EDGES = 160000
NEG_SLOPE = 0.2

NC, NS, LANES = 2, 16, 16
NW = NC * NS

_TYPE_OF = np.concatenate([np.full(NUMS[k], k, np.int32) for k in range(N_TYPES)])


def _mm_bias(a, b, bias):
    def body(a_ref, b_ref, c_ref, o_ref):
        o_ref[...] = jnp.dot(a_ref[...], b_ref[...],
                             preferred_element_type=jnp.float32) + c_ref[...]
    return pl.pallas_call(
        body,
        out_shape=jax.ShapeDtypeStruct((a.shape[0], b.shape[1]), jnp.float32),
    )(a, b, bias)


def _pack_rows(e, o):
    ue = lax.bitcast_convert_type(e, jnp.uint32)
    uo = lax.bitcast_convert_type(o, jnp.uint32)
    pe = (ue + jnp.uint32(0x8000)) >> 16
    po = (uo + jnp.uint32(0x8000)) & jnp.uint32(0xffff0000)
    return lax.bitcast_convert_type(pe | po, jnp.int32)


def _mm_pack(a, b, P, nextra):
    ncols = b.shape[1]

    def body(a_ref, b_ref, pk0, pk1, ex):
        x = jnp.dot(a_ref[...], b_ref[...], preferred_element_type=jnp.float32)
        pk0[...] = _pack_rows(x[0:P // 2], x[P // 2:P])
        pk1[...] = _pack_rows(x[P:P + P // 2], x[P + P // 2:2 * P])
        ex[...] = x[2 * P:2 * P + nextra]

    return pl.pallas_call(
        body,
        out_shape=(jax.ShapeDtypeStruct((P // 2, ncols), jnp.int32),
                   jax.ShapeDtypeStruct((P // 2, ncols), jnp.int32),
                   jax.ShapeDtypeStruct((nextra, ncols), jnp.float32)),
    )(a, b)


def _combine_hidden(acc0, acc1, dp0, dp1, b0, b1, wv, hres, H, D, use_res):
    P = H * D

    def body(*refs):
        if use_res:
            a0, a1, d0, d1, b0r, b1r, wvr, hr, o = refs
        else:
            a0, a1, d0, d1, b0r, b1r, wvr, o = refs
            hr = None
        outs = []
        for g, (a, d, br) in enumerate(((a0, d0, b0r), (a1, d1, b1r))):
            invs = []
            for h in range(H):
                den = jnp.sum(d[h * NW:(h + 1) * NW, :], axis=0, keepdims=True)
                invs.append(jnp.broadcast_to(1.0 / (den + 1e-16), (D, den.shape[1])))
            inv = jnp.concatenate(invs, axis=0)
            x = a[...] * inv + br[...]
            if hr is not None:
                x = x + hr[...]
            x = jnp.where(x > 0, x, jnp.exp(jnp.minimum(x, 0.0)) - 1.0)
            outs.append(x * jnp.broadcast_to(wvr[g:g + 1, :], x.shape))
        o[...] = outs[0] + outs[1]

    args = [acc0, acc1, dp0, dp1, b0, b1, wv] + ([hres] if use_res else [])
    return pl.pallas_call(
        body,
        out_shape=jax.ShapeDtypeStruct((P, N_NODES), jnp.float32),
    )(*args)


def _combine_final(acc_both, dp_both, b0, b1, res0, res1, wv):
    def body(ab, db, b0r, b1r, r0, r1, wvr, o):
        outs = []
        for g, (br, rr) in enumerate(((b0r, r0), (b1r, r1))):
            den = jnp.sum(db[g * NS:(g + 1) * NS, :], axis=0, keepdims=True)
            inv = jnp.broadcast_to(1.0 / (den + 1e-16), (NUM_CLASSES, den.shape[1]))
            acc = ab[g * NUM_CLASSES:(g + 1) * NUM_CLASSES, :]
            x = acc * inv + br[...] + rr[...]
            outs.append(x * jnp.broadcast_to(wvr[g:g + 1, :], x.shape))
        o[...] = outs[0] + outs[1]

    return pl.pallas_call(
        body,
        out_shape=jax.ShapeDtypeStruct((NUM_CLASSES, N_NODES), jnp.float32),
    )(acc_both, dp_both, b0, b1, res0, res1, wv)


def _sc_mesh():
    return plsc.VectorSubcoreMesh(core_axis_name="c", subcore_axis_name="s",
                                  num_cores=NC, num_subcores=NS)


_SC_PARAMS = pltpu.CompilerParams(needs_layout_passes=False)


@functools.partial(jax.jit, static_argnames=("H",))
def _sc_alpha(elr, src, dst, H):
    EPT = E_EDGES // NW
    EPTP = ((EPT + LANES - 1) // LANES) * LANES
    NIT = EPTP // LANES

    scratch = ([pltpu.VMEM((N_NODES,), jnp.float32)] * (3 * H)
               + [pltpu.VMEM((EPTP,), jnp.int32)] * 2
               + [pltpu.VMEM((EPTP,), jnp.float32)] * H
               + [pltpu.SemaphoreType.DMA])

    @functools.partial(
        pl.kernel,
        out_type=(jax.ShapeDtypeStruct((H * E_EDGES,), jnp.float32),
                  jax.ShapeDtypeStruct((H * NW * N_NODES,), jnp.float32)),
        mesh=_sc_mesh(),
        scratch_types=scratch,
        compiler_params=_SC_PARAMS,
    )
    def k(elr_h, src_h, dst_h, au_h, dp_h, *scr):
        el_loc = scr[0:H]
        er_loc = scr[H:2 * H]
        den = scr[2 * H:3 * H]
        srcb, dstb = scr[3 * H], scr[3 * H + 1]
        aub = scr[3 * H + 2:3 * H + 2 + H]
        sem = scr[3 * H + 2 + H]

        wid = lax.axis_index("c") * NS + lax.axis_index("s")
        base = wid * EPT
        copies = []
        for h in range(H):
            copies.append((elr_h.at[pl.ds(h * N_NODES, N_NODES)], el_loc[h]))
            copies.append((elr_h.at[pl.ds((H + h) * N_NODES, N_NODES)],
                           er_loc[h]))
        copies.append((src_h.at[pl.ds(base, EPT)], srcb.at[pl.ds(0, EPT)]))
        copies.append((dst_h.at[pl.ds(base, EPT)], dstb.at[pl.ds(0, EPT)]))
        for s_ref, d_ref in copies:
            pltpu.async_copy(s_ref, d_ref, sem)

        def zero_body(i, _):
            for h in range(H):
                den[h][pl.ds(i * LANES, LANES)] = jnp.zeros((LANES,), jnp.float32)
            return 0
        lax.fori_loop(0, N_NODES // LANES, zero_body, 0)

        for s_ref, d_ref in copies:
            pltpu.make_async_copy(s_ref, d_ref, sem).wait()

        if EPTP != EPT:
            lane = lax.iota(jnp.int32, LANES)
            keep = lane < (LANES - (EPTP - EPT))
            tail = pl.ds(EPTP - LANES, LANES)
            srcb[tail] = jnp.where(keep, srcb[tail], 0)
            dstb[tail] = jnp.where(keep, dstb[tail], 0)

        lane = lax.iota(jnp.int32, LANES)

        def body(i, _):
            kofs = i * LANES
            s16 = srcb[pl.ds(kofs, LANES)]
            d16 = dstb[pl.ds(kofs, LANES)]
            valid = (kofs + lane) < EPT
            for h in range(H):
                ev = plsc.load_gather(el_loc[h], [s16])
                rv = plsc.load_gather(er_loc[h], [d16])
                e = ev + rv
                e = jnp.where(e > 0, e, NEG_SLOPE * e)
                a = jnp.exp(e)
                aub[h][pl.ds(kofs, LANES)] = a
                plsc.addupdate_scatter(den[h], [d16], a, mask=valid)
            return 0
        lax.fori_loop(0, NIT, body, 0)

        for h in range(H):
            pltpu.sync_copy(aub[h].at[pl.ds(0, EPT)],
                            au_h.at[pl.ds(h * E_EDGES + base, EPT)])
            pltpu.sync_copy(den[h],
                            dp_h.at[pl.ds((h * NW + wid) * N_NODES, N_NODES)])

    return k(elr, src, dst)


@functools.partial(jax.jit, static_argnames=("H", "D", "esplit", "packed"))
def _sc_aggregate(featT, au, src, dst, H, D, esplit, packed):
    P = H * D
    PPT = P * esplit // NW
    NG = P // PPT
    EP = E_EDGES // esplit
    CH = 8000
    NCH = EP // CH
    UNROLL = 10
    NPHYS = PPT // 2 if packed else PPT
    fdt = jnp.int32 if packed else jnp.float32

    NACC = PPT
    scratch = (
        [pltpu.VMEM((N_NODES,), fdt) for _ in range(NPHYS)]
        + [pltpu.VMEM((N_NODES,), jnp.float32) for _ in range(NACC)]
        + [pltpu.VMEM((CH,), jnp.int32) for _ in range(2)]
        + [pltpu.VMEM((CH,), jnp.int32) for _ in range(2)]
        + [pltpu.VMEM((CH,), jnp.float32) for _ in range(2)]
        + [pltpu.SemaphoreType.DMA for _ in range(2)]
    )

    @functools.partial(
        pl.kernel,
        out_type=jax.ShapeDtypeStruct((esplit * P * N_NODES,), jnp.float32),
        mesh=_sc_mesh(),
        scratch_types=scratch,
        compiler_params=_SC_PARAMS,
    )
    def k(featT_h, au_h, src_h, dst_h, out_h, *scr):
        fps = scr[0:NPHYS]
        aps = scr[NPHYS:NPHYS + PPT]
        o = NPHYS + NACC
        srccs = scr[o:o + 2]
        dstcs = scr[o + 2:o + 4]
        aucs = scr[o + 4:o + 6]
        isems = scr[o + 6:o + 8]

        wid = lax.axis_index("c") * NS + lax.axis_index("s")
        g = wid % NG
        s = wid // NG
        pbase = g * PPT
        h = pbase // D
        fbase = g * NPHYS
        for p in range(NPHYS):
            pltpu.sync_copy(
                featT_h.at[pl.ds((fbase + p) * N_NODES, N_NODES)], fps[p])

        def zero_body(i, _):
            for r in aps:
                r[pl.ds(i * LANES, LANES)] = jnp.zeros((LANES,), jnp.float32)
            return 0
        lax.fori_loop(0, N_NODES // LANES, zero_body, 0)

        ebase = s * EP

        def issue_idx(c, b):
            off = ebase + c * CH
            pltpu.async_copy(src_h.at[pl.ds(off, CH)], srccs[b], isems[b])
            pltpu.async_copy(dst_h.at[pl.ds(off, CH)], dstcs[b], isems[b])
            pltpu.async_copy(au_h.at[pl.ds(h * E_EDGES + off, CH)],
                             aucs[b], isems[b])

        def drain_idx(c, b):
            off = ebase + c * CH
            pltpu.make_async_copy(src_h.at[pl.ds(off, CH)], srccs[b],
                                  isems[b]).wait()
            pltpu.make_async_copy(dst_h.at[pl.ds(off, CH)], dstcs[b],
                                  isems[b]).wait()
            pltpu.make_async_copy(au_h.at[pl.ds(h * E_EDGES + off, CH)],
                                  aucs[b], isems[b]).wait()

        issue_idx(0, 0)
        issue_idx(1, 1)

        def chunk_pair(j, _):
            for b in range(2):
                c = 2 * j + b
                drain_idx(c, b)

                def body(i, _):
                    for u in range(UNROLL):
                        kofs = i * (LANES * UNROLL) + u * LANES
                        s16 = srccs[b][pl.ds(kofs, LANES)]
                        d16 = dstcs[b][pl.ds(kofs, LANES)]
                        a16 = aucs[b][pl.ds(kofs, LANES)]
                        if packed:
                            tgt = aps
                            for p in range(NPHYS):
                                w = plsc.load_gather(fps[p], [s16])
                                ve = plsc.bitcast(w << 16, jnp.float32)
                                vo = plsc.bitcast(w & jnp.int32(-65536),
                                                  jnp.float32)
                                plsc.addupdate_scatter(tgt[2 * p], [d16],
                                                       ve * a16)
                                plsc.addupdate_scatter(tgt[2 * p + 1], [d16],
                                                       vo * a16)
                        else:
                            for p in range(PPT):
                                v = plsc.load_gather(fps[p], [s16])
                                plsc.addupdate_scatter(aps[p], [d16], v * a16)
                    return 0
                lax.fori_loop(0, CH // (LANES * UNROLL), body, 0)

                @pl.when(c + 2 < NCH)
                def _():
                    issue_idx(c + 2, b)
            return 0
        lax.fori_loop(0, NCH // 2, chunk_pair, 0)

        for p in range(PPT):
            pltpu.sync_copy(aps[p],
                            out_h.at[pl.ds((s * P + pbase + p) * N_NODES,
                                           N_NODES)])

    return k(featT, au, src, dst)


@jax.jit
def _sc_alpha_final(elr_both, src_both, dst_both):
    EPT = E_EDGES // NS
    NIT = EPT // LANES

    scratch = ([pltpu.VMEM((N_NODES,), jnp.float32)] * 3
               + [pltpu.VMEM((EPT,), jnp.int32)] * 2
               + [pltpu.VMEM((EPT,), jnp.float32)]
               + [pltpu.SemaphoreType.DMA])

    @functools.partial(
        pl.kernel,
        out_type=(jax.ShapeDtypeStruct((2 * E_EDGES,), jnp.float32),
                  jax.ShapeDtypeStruct((NW * N_NODES,), jnp.float32)),
        mesh=_sc_mesh(),
        scratch_types=scratch,
        compiler_params=_SC_PARAMS,
    )
    def k(elr_h, src_h, dst_h, au_h, dp_h, el_loc, er_loc, den,
          srcb, dstb, aub, sem):
        wid = lax.axis_index("c") * NS + lax.axis_index("s")
        g = wid // NS
        t = wid % NS
        base = g * E_EDGES + t * EPT
        copies = [
            (elr_h.at[pl.ds(g * 2 * N_NODES, N_NODES)], el_loc),
            (elr_h.at[pl.ds(g * 2 * N_NODES + N_NODES, N_NODES)], er_loc),
            (src_h.at[pl.ds(base, EPT)], srcb),
            (dst_h.at[pl.ds(base, EPT)], dstb),
        ]
        for s_ref, d_ref in copies:
            pltpu.async_copy(s_ref, d_ref, sem)

        def zero_body(i, _):
            den[pl.ds(i * LANES, LANES)] = jnp.zeros((LANES,), jnp.float32)
            return 0
        lax.fori_loop(0, N_NODES // LANES, zero_body, 0)

        for s_ref, d_ref in copies:
            pltpu.make_async_copy(s_ref, d_ref, sem).wait()

        def body(i, _):
            kofs = i * LANES
            s16 = srcb[pl.ds(kofs, LANES)]
            d16 = dstb[pl.ds(kofs, LANES)]
            e = plsc.load_gather(el_loc, [s16]) + plsc.load_gather(er_loc, [d16])
            e = jnp.where(e > 0, e, NEG_SLOPE * e)
            a = jnp.exp(e)
            aub[pl.ds(kofs, LANES)] = a
            plsc.addupdate_scatter(den, [d16], a)
            return 0
        lax.fori_loop(0, NIT, body, 0)

        pltpu.sync_copy(aub, au_h.at[pl.ds(base, EPT)])
        pltpu.sync_copy(den, dp_h.at[pl.ds(wid * N_NODES, N_NODES)])

    return k(elr_both, src_both, dst_both)


@jax.jit
def _sc_aggregate_final(featT_both, au_both, src_both, dst_both):
    P = NUM_CLASSES
    CH = 8000
    NCH = E_EDGES // CH

    scratch = (
        [pltpu.VMEM((N_NODES,), jnp.float32)] * 2
        + [pltpu.VMEM((CH,), jnp.int32) for _ in range(4)]
        + [pltpu.VMEM((CH,), jnp.float32) for _ in range(2)]
        + [pltpu.SemaphoreType.DMA for _ in range(2)]
    )

    @functools.partial(
        pl.kernel,
        out_type=jax.ShapeDtypeStruct((2 * P * N_NODES,), jnp.float32),
        mesh=_sc_mesh(),
        scratch_types=scratch,
        compiler_params=_SC_PARAMS,
    )
    def k(featT_h, au_h, src_h, dst_h, out_h, fp, ap,
          srcc0, srcc1, dstc0, dstc1, auc0, auc1, sem0, sem1):
        srccs, dstcs, aucs = (srcc0, srcc1), (dstc0, dstc1), (auc0, auc1)
        isems = (sem0, sem1)
        wid = lax.axis_index("c") * NS + lax.axis_index("s")
        g = wid // P
        p = wid % P
        pofs = (g * P + p) * N_NODES
        pltpu.sync_copy(featT_h.at[pl.ds(pofs, N_NODES)], fp)

        def zero_body(i, _):
            ap[pl.ds(i * LANES, LANES)] = jnp.zeros((LANES,), jnp.float32)
            return 0
        lax.fori_loop(0, N_NODES // LANES, zero_body, 0)

        ebase = g * E_EDGES

        def issue_idx(c, b):
            off = ebase + c * CH
            pltpu.async_copy(src_h.at[pl.ds(off, CH)], srccs[b], isems[b])
            pltpu.async_copy(dst_h.at[pl.ds(off, CH)], dstcs[b], isems[b])
            pltpu.async_copy(au_h.at[pl.ds(off, CH)], aucs[b], isems[b])

        def drain_idx(c, b):
            off = ebase + c * CH
            pltpu.make_async_copy(src_h.at[pl.ds(off, CH)], srccs[b],
                                  isems[b]).wait()
            pltpu.make_async_copy(dst_h.at[pl.ds(off, CH)], dstcs[b],
                                  isems[b]).wait()
            pltpu.make_async_copy(au_h.at[pl.ds(off, CH)], aucs[b],
                                  isems[b]).wait()

        issue_idx(0, 0)
        issue_idx(1, 1)

        def chunk_pair(j, _):
            for b in range(2):
                c = 2 * j + b
                drain_idx(c, b)

                def body(i, _):
                    for u in range(10):
                        kofs = i * (LANES * 10) + u * LANES
                        s16 = srccs[b][pl.ds(kofs, LANES)]
                        d16 = dstcs[b][pl.ds(kofs, LANES)]
                        a16 = aucs[b][pl.ds(kofs, LANES)]
                        v = plsc.load_gather(fp, [s16])
                        plsc.addupdate_scatter(ap, [d16], v * a16)
                    return 0
                lax.fori_loop(0, CH // (LANES * 10), body, 0)

                @pl.when(c + 2 < NCH)
                def _():
                    issue_idx(c + 2, b)
            return 0
        lax.fori_loop(0, NCH // 2, chunk_pair, 0)

        pltpu.sync_copy(ap, out_h.at[pl.ds(pofs, N_NODES)])

    return k(featT_both, au_both, src_both, dst_both)


def kernel(feat0, feat1, feat2, edge_index_g0, edge_index_g1, params):
    graphs = [(edge_index_g0[0].astype(jnp.int32), edge_index_g0[1].astype(jnp.int32)),
              (edge_index_g1[0].astype(jnp.int32), edge_index_g1[1].astype(jnp.int32))]
    w = jax.nn.softmax(params['rel_weights'], axis=2)
    type_of = jnp.asarray(_TYPE_OF)

    hs = []
    for i, f in enumerate((feat0, feat1, feat2)):
        p = params['fc'][i]
        hs.append(_mm_bias(p['W'].T, f.T, p['b'][:, None]))
    hT = jnp.concatenate(hs, axis=1)

    D = NUM_HIDDEN
    for l in range(NUM_LAYERS):
        H = HEADS[l]
        P = H * D
        perm = np.concatenate([np.arange(0, P, 2), np.arange(1, P, 2)])
        cols = []
        for i in range(N_GRAPHS):
            cols.append(params['gat'][i][l]['W'][:, perm])
        for i in range(N_GRAPHS):
            pg = params['gat'][i][l]
            al = [pg['W'][:, h * D:(h + 1) * D] @ pg['attn_l'][h] for h in range(H)]
            ar = [pg['W'][:, h * D:(h + 1) * D] @ pg['attn_r'][h] for h in range(H)]
            cols.append(jnp.stack(al + ar, axis=1))
        A = jnp.concatenate(cols, axis=1).T
        pk0, pk1, elr_all = _mm_pack(A, hT, P, 4 * H)
        pks = (pk0, pk1)
        accs, dps = [], []
        for i in range(N_GRAPHS):
            src, dst = graphs[i]
            elr = elr_all[2 * H * i:2 * H * (i + 1)]
            au, dp = _sc_alpha(elr.reshape(-1), src, dst, H)
            acc = _sc_aggregate(pks[i].reshape(-1), au, src, dst, H, D, 1,
                                packed=True)
            accs.append(acc.reshape(P, N_NODES))
            dps.append(dp.reshape(H * NW, N_NODES))
        wv = jnp.stack([w[type_of, l, i] for i in range(N_GRAPHS)])
        b0 = params['gat'][0][l]['bias'][:, None]
        b1 = params['gat'][1][l]['bias'][:, None]
        hT = _combine_hidden(accs[0], accs[1], dps[0], dps[1], b0, b1, wv,
                             hT, H, D, use_res=(l > 0))

    H, Df = HEADS[-1], NUM_CLASSES
    cols = []
    for i in range(N_GRAPHS):
        cols.append(params['gat'][i][NUM_LAYERS]['W'])
    for i in range(N_GRAPHS):
        cols.append(params['gat'][i][NUM_LAYERS]['res_W'])
    for i in range(N_GRAPHS):
        pg = params['gat'][i][NUM_LAYERS]
        cols.append(jnp.stack([pg['W'] @ pg['attn_l'][0],
                               pg['W'] @ pg['attn_r'][0]], axis=1))
    A = jnp.concatenate(cols, axis=1).T
    big = _mm_bias(A, hT, jnp.zeros((A.shape[0], 1), jnp.float32))
    src_both = jnp.concatenate([graphs[0][0], graphs[1][0]])
    dst_both = jnp.concatenate([graphs[0][1], graphs[1][1]])
    elr_both = big[4 * Df:4 * Df + 4].reshape(-1)
    au_both, dp = _sc_alpha_final(elr_both, src_both, dst_both)
    acc = _sc_aggregate_final(big[0:2 * Df].reshape(-1), au_both,
                              src_both, dst_both)
    wv = jnp.stack([w[type_of, NUM_LAYERS, i] for i in range(N_GRAPHS)])
    b0 = params['gat'][0][NUM_LAYERS]['bias'][:, None]
    b1 = params['gat'][1][NUM_LAYERS]['bias'][:, None]
    res0 = big[2 * Df:3 * Df]
    res1 = big[3 * Df:4 * Df]
    logitsT = _combine_final(acc.reshape(2 * Df, N_NODES),
                             dp.reshape(NW, N_NODES), b0, b1,
                             res0, res1, wv)
    return logitsT.T
